# Initial kernel scaffold; baseline (speedup 1.0000x reference)
import jax
import jax.numpy as jnp
from jax.experimental import pallas as pl


def kernel(points, normals, view_dirs, feature_vectors, phys_points, ray_dirs, cam_loc, W0, b0, W1, b1, W2, b2, W3, b3, W4, b4):
    raise NotImplementedError("write your pallas kernel here")


# speedup vs baseline: 8.4712x; 8.4712x over previous
import functools
import numpy as np

import jax
import jax.numpy as jnp
from jax.experimental import pallas as pl
from jax.experimental.pallas import tpu as pltpu

_N_RAYS = 512
_N_SAMP = 32
_N = _N_RAYS * _N_SAMP
_P = 4096
_FEAT = 256
_K = 20
_R2 = 81.0
_BLK = 256

_GROUPS = [
    (265, 3, 10),
    (328, 1, 4),
    (337, 3, 10),
    (400, 3, 10),
    (463, 3, 4),
    (490, 3, 4),
]
_NSIN = sum(d * f for _, d, f in _GROUPS)


def _stats_body(pts_ref, physT_ref, phys_ref, cam_ref, out_ref):
    pts = pts_ref[...]
    physT = physT_ref[...]
    phys = phys_ref[...]
    cam = cam_ref[...]

    qq = jnp.sum(pts * pts, axis=1, keepdims=True)
    pp = jnp.sum(physT * physT, axis=0, keepdims=True)
    cross = jnp.dot(pts, physT, preferred_element_type=jnp.float32)
    d2 = jnp.maximum(qq + pp - 2.0 * cross, 0.0)

    big = jnp.float32(np.inf)

    def sel_step(_, carry):
        m, cnt = carry
        v = jnp.min(jnp.where(d2 > m, d2, big), axis=1, keepdims=True)
        c = jnp.sum(jnp.where(d2 == v, 1.0, 0.0), axis=1, keepdims=True)
        take = cnt < float(_K)
        m = jnp.where(take, v, m)
        cnt = cnt + jnp.where(take, c, 0.0)
        return m, cnt

    m0 = jnp.full((pts.shape[0], 1), -big, jnp.float32)
    c0 = jnp.zeros((pts.shape[0], 1), jnp.float32)
    tau, _ = jax.lax.fori_loop(0, _K, sel_step, (m0, c0))

    sel = d2 <= tau
    sqrt_d2 = jnp.sqrt(d2)
    w = jnp.maximum(1.0 - d2 * sqrt_d2 * (1.0 / 729.0), 0.0)
    A = jnp.where(sel, w, 0.0)
    vmask = sel & (d2 < _R2)
    nnm = vmask & (d2 != 0.0)
    B = jnp.where(nnm, 1.0, 0.0)
    n_sel = jnp.sum(jnp.where(sel, 1.0, 0.0), axis=1, keepdims=True)
    n_sv = jnp.sum(jnp.where(vmask, 1.0, 0.0), axis=1, keepdims=True)

    ones = jnp.ones((phys.shape[0], 1), jnp.float32)
    PM = jnp.concatenate([phys, phys * phys, ones], axis=1)
    MA = jnp.dot(A, PM, precision=jax.lax.Precision.HIGHEST,
                 preferred_element_type=jnp.float32)
    MB = jnp.dot(B, PM, precision=jax.lax.Precision.HIGHEST,
                 preferred_element_type=jnp.float32)

    w_q = jnp.maximum(1.0 - qq * jnp.sqrt(qq) * (1.0 / 729.0), 0.0)
    density = MA[:, 6:7] + (n_sel - n_sv) * w_q
    smoothed = MA[:, 0:3] / (density + 1e-12)

    dirs = smoothed - cam
    sdir = dirs / jnp.sqrt(jnp.sum(dirs * dirs, axis=1, keepdims=True))

    num_nn = MB[:, 6:7]
    sp = MB[:, 0:3]
    sp2 = MB[:, 3:6]
    sdf = sp - num_nn * pts
    sd2f = sp2 - 2.0 * pts * sp + num_nn * pts * pts
    mean = sdf / (num_nn + 1e-12)
    var = (sd2f - 2.0 * mean * sdf + num_nn * mean * mean) / (num_nn + 1e-12)

    pad = jnp.zeros((pts.shape[0], 6), jnp.float32)
    out_ref[...] = jnp.concatenate([density, smoothed, var, sdir, pad], axis=1)


def _mlp_body(raw_ref, st_ref, mf_ref, wc_ref, b0_ref, w1_ref, b1_ref,
              w2_ref, b2_ref, w3_ref, b3_ref, w4_ref, b4_ref, out_ref):
    raw = raw_ref[...]
    st = st_ref[...]
    u_lin = jnp.concatenate([raw, st[:, 0:10]], axis=1)
    u16 = jnp.concatenate(
        [raw[:, 0:3], st[:, 0:1], st[:, 1:4], st[:, 4:7],
         raw[:, 265:268], st[:, 7:10]], axis=1)
    z = jnp.dot(u16, mf_ref[...], precision=jax.lax.Precision.HIGHEST,
                preferred_element_type=jnp.float32)
    x = jnp.concatenate([u_lin, jnp.sin(z), jnp.cos(z)], axis=1)
    h = jnp.dot(x, wc_ref[...], preferred_element_type=jnp.float32) + b0_ref[...]
    h = jnp.maximum(h, 0.0)
    h = jnp.dot(h, w1_ref[...], preferred_element_type=jnp.float32) + b1_ref[...]
    h = jnp.maximum(h, 0.0)
    h = jnp.dot(h, w2_ref[...], preferred_element_type=jnp.float32) + b2_ref[...]
    h = jnp.maximum(h, 0.0)
    h = jnp.dot(h, w3_ref[...], preferred_element_type=jnp.float32) + b3_ref[...]
    h = jnp.maximum(h, 0.0)
    h = jnp.dot(h, w4_ref[...], preferred_element_type=jnp.float32) + b4_ref[...]
    out_ref[...] = jax.nn.sigmoid(h)


def _embed_fold_indices():
    sin_idx, cos_idx = [], []
    for base, d, f in _GROUPS:
        for i in range(f):
            for c in range(d):
                sin_idx.append(base + d + 2 * i * d + c)
                cos_idx.append(base + d + (2 * i + 1) * d + c)
    return np.array(sin_idx), np.array(cos_idx)


def _freq_matrix():
    mf = np.zeros((16, _NSIN), np.float32)
    rows = [0, 3, 4, 7, 10, 13]
    col = 0
    for (base, d, f), r0 in zip(_GROUPS, rows):
        for i in range(f):
            for c in range(d):
                mf[r0 + c, col] = float(2.0 ** i)
                col += 1
    return mf


def kernel(points, normals, view_dirs, feature_vectors, phys_points, ray_dirs,
           cam_loc, W0, b0, W1, b1, W2, b2, W3, b3, W4, b4):
    ray_rep = jnp.repeat(ray_dirs, _N_SAMP, axis=0)
    raw = jnp.concatenate([points, view_dirs, normals, feature_vectors, ray_rep],
                          axis=1)
    physT = phys_points.T

    W0T = W0.T
    sin_idx, cos_idx = _embed_fold_indices()
    lin = jnp.concatenate([
        W0T[0:3] + W0T[265:268],
        W0T[3:9],
        W0T[9:265],
        W0T[463:466],
        W0T[328:329],
        W0T[337:340],
        W0T[400:403],
        W0T[490:493],
    ], axis=0)
    Wcat = jnp.concatenate([lin, W0T[sin_idx], W0T[cos_idx]], axis=0)
    Mf = jnp.asarray(_freq_matrix())

    grid = (_N // _BLK,)
    stats = pl.pallas_call(
        _stats_body,
        grid=grid,
        in_specs=[
            pl.BlockSpec((_BLK, 3), lambda i: (i, 0)),
            pl.BlockSpec((3, _P), lambda i: (0, 0)),
            pl.BlockSpec((_P, 3), lambda i: (0, 0)),
            pl.BlockSpec((1, 3), lambda i: (0, 0)),
        ],
        out_specs=pl.BlockSpec((_BLK, 16), lambda i: (i, 0)),
        out_shape=jax.ShapeDtypeStruct((_N, 16), jnp.float32),
    )(points, physT, phys_points, cam_loc)

    full = lambda s: pl.BlockSpec(s, lambda i: tuple(0 for _ in s))
    out = pl.pallas_call(
        _mlp_body,
        grid=grid,
        in_specs=[
            pl.BlockSpec((_BLK, 268), lambda i: (i, 0)),
            pl.BlockSpec((_BLK, 16), lambda i: (i, 0)),
            full((16, _NSIN)),
            full((514, 512)),
            full((1, 512)),
            full((512, 512)),
            full((1, 512)),
            full((512, 512)),
            full((1, 512)),
            full((512, 512)),
            full((1, 512)),
            full((512, 3)),
            full((1, 3)),
        ],
        out_specs=pl.BlockSpec((_BLK, 3), lambda i: (i, 0)),
        out_shape=jax.ShapeDtypeStruct((_N, 3), jnp.float32),
    )(raw, stats, Mf, Wcat, b0.reshape(1, -1), W1.T, b1.reshape(1, -1),
      W2.T, b2.reshape(1, -1), W3.T, b3.reshape(1, -1), W4.T, b4.reshape(1, -1))
    return out


# speedup vs baseline: 10.8374x; 1.2793x over previous
import functools
import numpy as np

import jax
import jax.numpy as jnp
from jax.experimental import pallas as pl
from jax.experimental.pallas import tpu as pltpu

_N_RAYS = 512
_N_SAMP = 32
_N = _N_RAYS * _N_SAMP
_P = 4096
_FEAT = 256
_K = 20
_R2 = 81.0
_BLK = 256

_GROUPS = [
    (265, 3, 10),
    (328, 1, 4),
    (337, 3, 10),
    (400, 3, 10),
    (463, 3, 4),
    (490, 3, 4),
]
_NSIN = sum(d * f for _, d, f in _GROUPS)


def _stats_body(pts_ref, physT_ref, phys_ref, cam_ref, out_ref):
    pts = pts_ref[...]
    physT = physT_ref[...]
    phys = phys_ref[...]
    cam = cam_ref[...]

    qq = jnp.sum(pts * pts, axis=1, keepdims=True)
    pp = jnp.sum(physT * physT, axis=0, keepdims=True)
    cross = jnp.dot(pts, physT, preferred_element_type=jnp.float32)
    d2 = jnp.maximum(qq + pp - 2.0 * cross, 0.0)

    big = jnp.float32(np.inf)
    kf = float(_K)

    hi0 = jnp.max(d2, axis=1, keepdims=True)
    n_zero = jnp.sum(jnp.where(d2 <= 0.0, 1.0, 0.0), axis=1, keepdims=True)
    lo0 = jnp.zeros_like(hi0)

    def bis_step(_, carry):
        lo, clo, hi = carry
        mid = 0.5 * (lo + hi)
        c = jnp.sum(jnp.where(d2 <= mid, 1.0, 0.0), axis=1, keepdims=True)
        ge = c >= kf
        return (jnp.where(ge, lo, mid), jnp.where(ge, clo, c),
                jnp.where(ge, mid, hi))

    lo, clo, _ = jax.lax.fori_loop(0, 16, bis_step, (lo0, n_zero, hi0))

    def sel_step(_, carry):
        m, cnt = carry
        v = jnp.min(jnp.where(d2 > m, d2, big), axis=1, keepdims=True)
        c = jnp.sum(jnp.where(d2 <= v, 1.0, 0.0), axis=1, keepdims=True)
        take = cnt < kf
        return jnp.where(take, v, m), jnp.where(take, c, cnt)

    tau, n_sel = jax.lax.fori_loop(0, 5, sel_step, (lo, clo))

    sel = d2 <= tau
    dx = physT[0:1, :] - pts[:, 0:1]
    dy = physT[1:2, :] - pts[:, 1:2]
    dz = physT[2:3, :] - pts[:, 2:3]
    d2t = dx * dx + dy * dy + dz * dz
    w = jnp.maximum(1.0 - d2t * jnp.sqrt(d2t) * (1.0 / 729.0), 0.0)
    A = jnp.where(sel, w, 0.0)
    nnm = sel & (d2 < _R2) & (d2 != 0.0)

    s_w = jnp.sum(A, axis=1, keepdims=True)
    s_wp = jnp.concatenate(
        [jnp.sum(A * physT[c:c + 1, :], axis=1, keepdims=True) for c in range(3)],
        axis=1)

    num_nn = jnp.sum(jnp.where(nnm, 1.0, 0.0), axis=1, keepdims=True)
    sdf_c, sd2f_c = [], []
    for dc in (dx, dy, dz):
        dcm = jnp.where(nnm, dc, 0.0)
        sdf_c.append(jnp.sum(dcm, axis=1, keepdims=True))
        sd2f_c.append(jnp.sum(dcm * dcm, axis=1, keepdims=True))
    sdf = jnp.concatenate(sdf_c, axis=1)
    sd2f = jnp.concatenate(sd2f_c, axis=1)

    w_q = jnp.maximum(1.0 - qq * jnp.sqrt(qq) * (1.0 / 729.0), 0.0)
    n_sv = num_nn + n_zero
    density = s_w + (n_sel - n_sv) * w_q
    smoothed = s_wp / (density + 1e-12)

    dirs = smoothed - cam
    sdir = dirs / jnp.sqrt(jnp.sum(dirs * dirs, axis=1, keepdims=True))

    mean = sdf / (num_nn + 1e-12)
    var = (sd2f - 2.0 * mean * sdf + num_nn * mean * mean) / (num_nn + 1e-12)

    pad = jnp.zeros((pts.shape[0], 6), jnp.float32)
    out_ref[...] = jnp.concatenate([density, smoothed, var, sdir, pad], axis=1)


def _mlp_body(raw_ref, st_ref, mf_ref, wc_ref, b0_ref, w1_ref, b1_ref,
              w2_ref, b2_ref, w3_ref, b3_ref, w4_ref, b4_ref, out_ref):
    raw = raw_ref[...]
    st = st_ref[...]
    u_lin = jnp.concatenate([raw, st[:, 0:10], raw[:, 0:3]], axis=1)
    u16 = jnp.concatenate(
        [raw[:, 0:3], st[:, 0:1], st[:, 1:4], st[:, 4:7],
         raw[:, 265:268], st[:, 7:10]], axis=1)
    z = jnp.dot(u16, mf_ref[...], precision=jax.lax.Precision.HIGHEST,
                preferred_element_type=jnp.float32)
    x = jnp.concatenate([u_lin, jnp.sin(z), jnp.cos(z)], axis=1)
    h = jnp.dot(x, wc_ref[...], preferred_element_type=jnp.float32) + b0_ref[...]
    h = jnp.maximum(h, 0.0)
    h = jnp.dot(h, w1_ref[...], preferred_element_type=jnp.float32) + b1_ref[...]
    h = jnp.maximum(h, 0.0)
    h = jnp.dot(h, w2_ref[...], preferred_element_type=jnp.float32) + b2_ref[...]
    h = jnp.maximum(h, 0.0)
    h = jnp.dot(h, w3_ref[...], preferred_element_type=jnp.float32) + b3_ref[...]
    h = jnp.maximum(h, 0.0)
    h = jnp.dot(h, w4_ref[...], preferred_element_type=jnp.float32) + b4_ref[...]
    out_ref[...] = jax.nn.sigmoid(h)


def _embed_fold_indices():
    sin_idx, cos_idx = [], []
    for base, d, f in _GROUPS:
        for i in range(f):
            for c in range(d):
                sin_idx.append(base + d + 2 * i * d + c)
                cos_idx.append(base + d + (2 * i + 1) * d + c)
    return np.array(sin_idx), np.array(cos_idx)


def _freq_matrix():
    mf = np.zeros((16, _NSIN), np.float32)
    rows = [0, 3, 4, 7, 10, 13]
    col = 0
    for (base, d, f), r0 in zip(_GROUPS, rows):
        for i in range(f):
            for c in range(d):
                mf[r0 + c, col] = float(2.0 ** i)
                col += 1
    return mf


def kernel(points, normals, view_dirs, feature_vectors, phys_points, ray_dirs,
           cam_loc, W0, b0, W1, b1, W2, b2, W3, b3, W4, b4):
    ray_rep = jnp.repeat(ray_dirs, _N_SAMP, axis=0)
    raw = jnp.concatenate([points, view_dirs, normals, feature_vectors, ray_rep],
                          axis=1)
    physT = phys_points.T

    W0T = W0.T
    sin_idx, cos_idx = _embed_fold_indices()
    lin = jnp.concatenate([
        W0T[0:3],
        W0T[3:9],
        W0T[9:265],
        W0T[463:466],
        W0T[328:329],
        W0T[337:340],
        W0T[400:403],
        W0T[490:493],
        W0T[265:268],
    ], axis=0)
    Wcat = jnp.concatenate([lin, W0T[sin_idx], W0T[cos_idx]], axis=0)
    Mf = jnp.asarray(_freq_matrix())

    grid = (_N // _BLK,)
    stats = pl.pallas_call(
        _stats_body,
        grid=grid,
        in_specs=[
            pl.BlockSpec((_BLK, 3), lambda i: (i, 0)),
            pl.BlockSpec((3, _P), lambda i: (0, 0)),
            pl.BlockSpec((_P, 3), lambda i: (0, 0)),
            pl.BlockSpec((1, 3), lambda i: (0, 0)),
        ],
        out_specs=pl.BlockSpec((_BLK, 16), lambda i: (i, 0)),
        out_shape=jax.ShapeDtypeStruct((_N, 16), jnp.float32),
    )(points, physT, phys_points, cam_loc)

    full = lambda s: pl.BlockSpec(s, lambda i: tuple(0 for _ in s))
    out = pl.pallas_call(
        _mlp_body,
        grid=grid,
        in_specs=[
            pl.BlockSpec((_BLK, 268), lambda i: (i, 0)),
            pl.BlockSpec((_BLK, 16), lambda i: (i, 0)),
            full((16, _NSIN)),
            full((517, 512)),
            full((1, 512)),
            full((512, 512)),
            full((1, 512)),
            full((512, 512)),
            full((1, 512)),
            full((512, 512)),
            full((1, 512)),
            full((512, 3)),
            full((1, 3)),
        ],
        out_specs=pl.BlockSpec((_BLK, 3), lambda i: (i, 0)),
        out_shape=jax.ShapeDtypeStruct((_N, 3), jnp.float32),
    )(raw, stats, Mf, Wcat, b0.reshape(1, -1), W1.T, b1.reshape(1, -1),
      W2.T, b2.reshape(1, -1), W3.T, b3.reshape(1, -1), W4.T, b4.reshape(1, -1))
    return out
